# baseline (speedup 1.0000x reference)
import functools

import jax
import jax.numpy as jnp
from jax import lax
from jax.experimental import pallas as pl
from jax.experimental.pallas import tpu as pltpu
from jax.experimental.pallas import tpu_sc as plsc

N = 10000
E = 320000
L = 128
NP = 10240
EP = 327680
ROWS = 160
STRIPE = 640
NSTEPS = 10
F32 = jnp.float32
I32 = jnp.int32


def _sc_sinkhorn(y_p, src_p, dst_p):
    mesh = plsc.VectorSubcoreMesh(core_axis_name="c", subcore_axis_name="s")

    @functools.partial(
        pl.kernel,
        out_type=jax.ShapeDtypeStruct((16, ROWS, 128), F32),
        mesh=mesh,
        compiler_params=pltpu.CompilerParams(needs_layout_passes=False),
        scratch_types=[
            pltpu.VMEM((ROWS, 128), I32),
            pltpu.VMEM((ROWS, 128), I32),
            pltpu.VMEM((ROWS, 128), F32),
            pltpu.VMEM((NP,), F32),
            pltpu.VMEM((STRIPE,), F32),
            pltpu.VMEM((STRIPE,), F32),
            pltpu.SemaphoreType.DMA,
            pltpu.VMEM_SHARED((NP,), F32),
            pltpu.VMEM_SHARED((NP,), F32),
        ],
    )
    def k(y_hbm, src_hbm, dst_hbm, w_hbm,
          src2, dst2, w2, lbuf, lnbuf, zbuf, esem, S_sh, R_sh):
        sid = lax.axis_index("s")
        sbase = sid * STRIPE

        pltpu.sync_copy(y_hbm.at[sid], w2)
        pltpu.sync_copy(src_hbm.at[sid], src2)
        pltpu.sync_copy(dst_hbm.at[sid], dst2)

        @pl.loop(0, STRIPE, step=16)
        def _(i):
            zbuf[pl.ds(i, 16)] = jnp.zeros((16,), F32)

        pltpu.sync_copy(zbuf, S_sh.at[pl.ds(sbase, STRIPE)])

        @pl.loop(0, ROWS)
        def _(r):
            for j in range(0, 128, 16):
                yv = w2[r, pl.ds(j, 16)]
                sv = src2[r, pl.ds(j, 16)]
                dv = dst2[r, pl.ds(j, 16)]
                yv = jnp.where(sv == dv, F32(-1.0e5), yv) / F32(0.1)
                w2[r, pl.ds(j, 16)] = jnp.exp(yv)

        plsc.subcore_barrier()

        LAG = 32

        def half_pass(idx2):
            @pl.loop(0, ROWS)
            def _(r):
                pltpu.async_copy(w2.at[r], S_sh.at[idx2.at[r]], esem,
                                 add=True)

                @pl.when(r >= LAG)
                def _():
                    pltpu.make_async_copy(y_hbm.at[sid, 0],
                                          w2.at[r - LAG], esem).wait()

            @pl.loop(ROWS - LAG, ROWS)
            def _(r):
                pltpu.make_async_copy(y_hbm.at[sid, 0], w2.at[r],
                                      esem).wait()

            plsc.subcore_barrier()

            pltpu.sync_copy(S_sh.at[pl.ds(sbase, STRIPE)], lnbuf)

            @pl.loop(0, STRIPE, step=16)
            def _(i):
                lnbuf[pl.ds(i, 16)] = F32(1.0) / (lnbuf[pl.ds(i, 16)]
                                                  + F32(1e-37))

            pltpu.sync_copy(lnbuf, R_sh.at[pl.ds(sbase, STRIPE)])
            plsc.subcore_barrier()

            pltpu.sync_copy(R_sh, lbuf)
            pltpu.sync_copy(zbuf, S_sh.at[pl.ds(sbase, STRIPE)])

            @pl.loop(0, ROWS)
            def _(r):
                for j in range(0, 128, 16):
                    idxv = idx2[r, pl.ds(j, 16)]
                    g = plsc.load_gather(lbuf, [idxv])
                    w2[r, pl.ds(j, 16)] = w2[r, pl.ds(j, 16)] * g

            plsc.subcore_barrier()

        @pl.loop(0, NSTEPS)
        def _(_step):
            half_pass(src2)
            half_pass(dst2)

        @pl.when(lax.axis_index("c") == 0)
        def _():
            pltpu.sync_copy(w2, w_hbm.at[sid])

    return k(y_p, src_p, dst_p)


def _sc_message(w_p, src_p, dst_p, a_nd, b_nd):
    mesh = plsc.VectorSubcoreMesh(core_axis_name="c", subcore_axis_name="s")
    CH = 64

    NR = ROWS // 2

    @functools.partial(
        pl.kernel,
        out_type=jax.ShapeDtypeStruct((2, NP, L), F32),
        mesh=mesh,
        compiler_params=pltpu.CompilerParams(needs_layout_passes=False),
        scratch_types=[
            pltpu.VMEM((2, 128), I32),
            pltpu.VMEM((2, 128), I32),
            pltpu.VMEM((2, 128), F32),
            pltpu.VMEM((2, CH), I32),
            pltpu.VMEM((2, CH, L), F32),
            pltpu.VMEM((2, CH, L), F32),
            pltpu.VMEM((CH, L), F32),
            pltpu.SemaphoreType.DMA((2,)),
            pltpu.SemaphoreType.DMA((2,)),
            pltpu.SemaphoreType.DMA,
            pltpu.VMEM_SHARED((NP, L), F32),
        ],
    )
    def k(w_hbm, src_hbm, dst_hbm, a_hbm, b_hbm, out_hbm,
          sb, db, wb, dsc, gA, gB, mbuf, isem, gsem, ssem, agg_sh):
        cid = lax.axis_index("c")
        sid = lax.axis_index("s")
        sbase = sid * STRIPE
        rbase = cid * NR

        def issue_idxrow(k, p):
            row = rbase + k
            pltpu.async_copy(src_hbm.at[sid, row], sb.at[p], isem.at[p])
            pltpu.async_copy(dst_hbm.at[sid, row], db.at[p], isem.at[p])
            pltpu.async_copy(w_hbm.at[sid, row], wb.at[p], isem.at[p])

        def wait_idxrow(p):
            pltpu.make_async_copy(src_hbm.at[sid, 0], sb.at[p],
                                  isem.at[p]).wait()
            pltpu.make_async_copy(src_hbm.at[sid, 0], db.at[p],
                                  isem.at[p]).wait()
            pltpu.make_async_copy(w_hbm.at[sid, 0], wb.at[p],
                                  isem.at[p]).wait()

        def issue_gathers(p, h):
            ds = pl.ds(h * CH, CH)
            pltpu.async_copy(a_hbm.at[sb.at[p, ds]], gA.at[h], gsem.at[h])
            pltpu.async_copy(b_hbm.at[db.at[p, ds]], gB.at[h], gsem.at[h])

        def wait_gathers(h):
            pltpu.make_async_copy(a_hbm.at[pl.ds(0, CH)], gA.at[h],
                                  gsem.at[h]).wait()
            pltpu.make_async_copy(a_hbm.at[pl.ds(0, CH)], gB.at[h],
                                  gsem.at[h]).wait()

        def wait_scatter():
            pltpu.make_async_copy(a_hbm.at[pl.ds(0, CH)], mbuf, ssem).wait()

        def compute(p, h):
            for i in range(0, CH, 16):
                dsc[h, pl.ds(i, 16)] = db[p, pl.ds(h * CH + i, 16)]

            @pl.loop(0, CH, step=16)
            def _(qb):
                w16 = wb[p, pl.ds(h * CH + qb, 16)]
                for t in range(16):
                    q = qb + t
                    w = w16[t]
                    for j in range(0, L, 16):
                        va = gA[h, q, pl.ds(j, 16)]
                        vb = gB[h, q, pl.ds(j, 16)]
                        mbuf[q, pl.ds(j, 16)] = (
                            jnp.maximum(va + vb, F32(0.0)) * w)

            pltpu.async_copy(mbuf, agg_sh.at[dsc.at[h]], ssem, add=True)

        issue_idxrow(0, 0)

        @pl.loop(0, CH)
        def _(q):
            for j in range(0, L, 16):
                mbuf[q, pl.ds(j, 16)] = jnp.zeros((16,), F32)

        wait_idxrow(0)
        issue_gathers(0, 0)

        @pl.loop(0, STRIPE, step=CH)
        def _(i):
            pltpu.sync_copy(mbuf, agg_sh.at[pl.ds(sbase + i, CH)])

        plsc.subcore_barrier()

        @pl.loop(0, NR // 2)
        def _(kk):
            for rp in range(2):
                k = 2 * kk + rp

                if rp == 0:
                    @pl.when(kk > 0)
                    def _():
                        wait_scatter()
                else:
                    wait_scatter()
                if rp == 0:
                    issue_idxrow(k + 1, 1)
                else:
                    @pl.when(kk < NR // 2 - 1)
                    def _():
                        issue_idxrow(k + 1, 0)
                issue_gathers(rp, 1)
                wait_gathers(0)
                compute(rp, 0)

                wait_scatter()
                if rp == 0:
                    wait_idxrow(1)
                    issue_gathers(1, 0)
                else:
                    @pl.when(kk < NR // 2 - 1)
                    def _():
                        wait_idxrow(0)
                        issue_gathers(0, 0)
                wait_gathers(1)
                compute(rp, 1)

        wait_scatter()
        plsc.subcore_barrier()
        pltpu.sync_copy(agg_sh.at[pl.ds(sbase, STRIPE)],
                        out_hbm.at[cid, pl.ds(sbase, STRIPE)])

    return k(w_p, src_p, dst_p, a_nd, b_nd)


def _dot_t(a, w):
    return lax.dot_general(a, w, (((1,), (1,)), ((), ())),
                           preferred_element_type=F32,
                           precision=lax.Precision.HIGHEST)


def _enc_body(x_ref, hid_ref, we_ref, be_ref, wm_ref, bm_ref,
              h_ref, a_ref, b_ref):
    we = we_ref[...]
    h = _dot_t(x_ref[...], we[:, :L]) + _dot_t(hid_ref[...], we[:, L:])
    h = jnp.maximum(h + be_ref[...], F32(0.0))
    h_ref[...] = h
    wm = wm_ref[...]
    a_ref[...] = _dot_t(h, wm[:, :L])
    b_ref[...] = _dot_t(h, wm[:, L:]) + bm_ref[...]


def _upd_body(h_ref, p0_ref, p1_ref, wu_ref, bu_ref, o_ref):
    wu = wu_ref[...]
    agg = p0_ref[...] + p1_ref[...]
    o = _dot_t(h_ref[...], wu[:, :L]) + _dot_t(agg, wu[:, L:])
    o_ref[...] = jnp.maximum(o + bu_ref[...], F32(0.0))


_ROWBLK = 640


def _row_spec():
    return pl.BlockSpec((_ROWBLK, L), lambda i: (i, 0))


def _full_spec(shape):
    nd = len(shape)
    return pl.BlockSpec(shape, lambda i: (0,) * nd)


def _encode(xp, hp, W_enc, b_enc, W_msg, b_msg):
    grid = (NP // _ROWBLK,)
    sds = jax.ShapeDtypeStruct((NP, L), F32)
    return pl.pallas_call(
        _enc_body,
        grid=grid,
        in_specs=[_row_spec(), _row_spec(),
                  _full_spec((L, 2 * L)), _full_spec((1, L)),
                  _full_spec((L, 2 * L)), _full_spec((1, L))],
        out_specs=[_row_spec(), _row_spec(), _row_spec()],
        out_shape=[sds, sds, sds],
    )(xp, hp, W_enc, b_enc, W_msg, b_msg)


def _update(h, p0, p1, W_upd, b_upd):
    grid = (NP // _ROWBLK,)
    return pl.pallas_call(
        _upd_body,
        grid=grid,
        in_specs=[_row_spec(), _row_spec(), _row_spec(),
                  _full_spec((L, 2 * L)), _full_spec((1, L))],
        out_specs=_row_spec(),
        out_shape=jax.ShapeDtypeStruct((NP, L), F32),
    )(h, p0, p1, W_upd, b_upd)


def kernel(x, hidden, y, edge_index, W_enc, b_enc, W_msg, b_msg, W_upd, b_upd):
    src = edge_index[0].astype(I32)
    dst = edge_index[1].astype(I32)
    pad = EP - E
    pad_idx = (jnp.arange(pad, dtype=I32) * 13) % N
    src_p = jnp.concatenate([src, pad_idx])
    dst_p = jnp.concatenate([dst, pad_idx])
    y_p = jnp.concatenate([y, jnp.zeros((pad,), F32)])
    src_p = src_p.reshape(16, ROWS, 128)
    dst_p = dst_p.reshape(16, ROWS, 128)
    y_p = y_p.reshape(16, ROWS, 128)

    zpad = jnp.zeros((NP - N, L), F32)
    xp = jnp.concatenate([x, zpad])
    hp = jnp.concatenate([hidden, zpad])

    h, a_nd, b_nd = _encode(xp, hp, W_enc, b_enc.reshape(1, L),
                            W_msg, b_msg.reshape(1, L))
    w_p = _sc_sinkhorn(y_p, src_p, dst_p)
    parts = _sc_message(w_p, src_p, dst_p, a_nd, b_nd)
    h_new = _update(h, parts[0], parts[1], W_upd, b_upd.reshape(1, L))
    return h_new[:N]


# speedup vs baseline: 1.0420x; 1.0420x over previous
import functools

import jax
import jax.numpy as jnp
from jax import lax
from jax.experimental import pallas as pl
from jax.experimental.pallas import tpu as pltpu
from jax.experimental.pallas import tpu_sc as plsc

N = 10000
E = 320000
L = 128
NP = 10240
EP = 327680
ROWS = 160
STRIPE = 640
NSTEPS = 10
F32 = jnp.float32
I32 = jnp.int32


def _sc_sinkhorn(y_p, src_p, dst_p):
    mesh = plsc.VectorSubcoreMesh(core_axis_name="c", subcore_axis_name="s")

    @functools.partial(
        pl.kernel,
        out_type=jax.ShapeDtypeStruct((16, ROWS, 128), F32),
        mesh=mesh,
        compiler_params=pltpu.CompilerParams(needs_layout_passes=False),
        scratch_types=[
            pltpu.VMEM((ROWS, 128), I32),
            pltpu.VMEM((ROWS, 128), I32),
            pltpu.VMEM((ROWS, 128), F32),
            pltpu.VMEM((NP,), F32),
            pltpu.VMEM((STRIPE,), F32),
            pltpu.VMEM((STRIPE,), F32),
            pltpu.SemaphoreType.DMA,
            pltpu.VMEM_SHARED((NP,), F32),
            pltpu.VMEM_SHARED((NP,), F32),
        ],
    )
    def k(y_hbm, src_hbm, dst_hbm, w_hbm,
          src2, dst2, w2, lbuf, lnbuf, zbuf, esem, S_sh, R_sh):
        sid = lax.axis_index("s")
        sbase = sid * STRIPE

        pltpu.sync_copy(y_hbm.at[sid], w2)
        pltpu.sync_copy(src_hbm.at[sid], src2)
        pltpu.sync_copy(dst_hbm.at[sid], dst2)

        @pl.loop(0, STRIPE, step=16)
        def _(i):
            zbuf[pl.ds(i, 16)] = jnp.zeros((16,), F32)

        pltpu.sync_copy(zbuf, S_sh.at[pl.ds(sbase, STRIPE)])

        @pl.loop(0, ROWS)
        def _(r):
            for j in range(0, 128, 16):
                yv = w2[r, pl.ds(j, 16)]
                sv = src2[r, pl.ds(j, 16)]
                dv = dst2[r, pl.ds(j, 16)]
                yv = jnp.where(sv == dv, F32(-1.0e5), yv) / F32(0.1)
                w2[r, pl.ds(j, 16)] = jnp.exp(yv)

        plsc.subcore_barrier()

        LAG = 32

        def half_pass(idx2):
            @pl.loop(0, ROWS)
            def _(r):
                pltpu.async_copy(w2.at[r], S_sh.at[idx2.at[r]], esem,
                                 add=True)

                @pl.when(r >= LAG)
                def _():
                    pltpu.make_async_copy(y_hbm.at[sid, 0],
                                          w2.at[r - LAG], esem).wait()

            @pl.loop(ROWS - LAG, ROWS)
            def _(r):
                pltpu.make_async_copy(y_hbm.at[sid, 0], w2.at[r],
                                      esem).wait()

            plsc.subcore_barrier()

            pltpu.sync_copy(S_sh.at[pl.ds(sbase, STRIPE)], lnbuf)

            @pl.loop(0, STRIPE, step=16)
            def _(i):
                lnbuf[pl.ds(i, 16)] = F32(1.0) / (lnbuf[pl.ds(i, 16)]
                                                  + F32(1e-37))

            pltpu.sync_copy(lnbuf, R_sh.at[pl.ds(sbase, STRIPE)])
            plsc.subcore_barrier()

            pltpu.sync_copy(R_sh, lbuf)
            pltpu.sync_copy(zbuf, S_sh.at[pl.ds(sbase, STRIPE)])

            @pl.loop(0, ROWS)
            def _(r):
                for j in range(0, 128, 16):
                    idxv = idx2[r, pl.ds(j, 16)]
                    g = plsc.load_gather(lbuf, [idxv])
                    w2[r, pl.ds(j, 16)] = w2[r, pl.ds(j, 16)] * g

            plsc.subcore_barrier()

        @pl.loop(0, NSTEPS)
        def _(_step):
            half_pass(src2)
            half_pass(dst2)

        @pl.when(lax.axis_index("c") == 0)
        def _():
            pltpu.sync_copy(w2, w_hbm.at[sid])

    return k(y_p, src_p, dst_p)


def _sc_message(w_p, src_p, dst_p, a_nd, b_nd):
    mesh = plsc.VectorSubcoreMesh(core_axis_name="c", subcore_axis_name="s")
    CH = 64

    @functools.partial(
        pl.kernel,
        out_type=jax.ShapeDtypeStruct((2, NP, L), F32),
        mesh=mesh,
        compiler_params=pltpu.CompilerParams(needs_layout_passes=False),
        scratch_types=[
            pltpu.VMEM((2, CH), I32),
            pltpu.VMEM((2, CH), I32),
            pltpu.VMEM((2, CH), F32),
            pltpu.VMEM((2, CH, L), F32),
            pltpu.VMEM((2, CH, L), F32),
            pltpu.VMEM((CH, L), F32),
            pltpu.SemaphoreType.DMA,
            pltpu.SemaphoreType.DMA((2,)),
            pltpu.SemaphoreType.DMA,
            pltpu.VMEM_SHARED((NP, L), F32),
        ],
    )
    def k(w_hbm, src_hbm, dst_hbm, a_hbm, b_hbm, out_hbm,
          sb, db, wb, gA, gB, mbuf, isem, gsem, ssem, agg_sh):
        cid = lax.axis_index("c")
        sid = lax.axis_index("s")
        sbase = sid * STRIPE

        @pl.loop(0, CH)
        def _(q):
            for j in range(0, L, 16):
                mbuf[q, pl.ds(j, 16)] = jnp.zeros((16,), F32)

        @pl.loop(0, STRIPE, step=CH)
        def _(i):
            pltpu.sync_copy(mbuf, agg_sh.at[pl.ds(sbase + i, CH)])

        plsc.subcore_barrier()

        rbase = cid * (ROWS // 2)

        def issue_idx(row, half, slot):
            ds = pl.ds(half * CH, CH)
            d1 = pltpu.async_copy(src_hbm.at[sid, row, ds], sb.at[slot], isem)
            d2 = pltpu.async_copy(dst_hbm.at[sid, row, ds], db.at[slot], isem)
            d3 = pltpu.async_copy(w_hbm.at[sid, row, ds], wb.at[slot], isem)
            d1.wait()
            d2.wait()
            d3.wait()

        def issue_gathers(slot):
            pltpu.async_copy(a_hbm.at[sb.at[slot]], gA.at[slot],
                             gsem.at[slot])
            pltpu.async_copy(b_hbm.at[db.at[slot]], gB.at[slot],
                             gsem.at[slot])

        def wait_gathers(slot):
            pltpu.make_async_copy(a_hbm.at[pl.ds(0, CH)], gA.at[slot],
                                  gsem.at[slot]).wait()
            pltpu.make_async_copy(a_hbm.at[pl.ds(0, CH)], gB.at[slot],
                                  gsem.at[slot]).wait()

        def wait_scatter():
            pltpu.make_async_copy(a_hbm.at[pl.ds(0, CH)], mbuf, ssem).wait()

        issue_idx(rbase, 0, 0)
        issue_gathers(0)

        @pl.loop(0, ROWS // 2)
        def _(kk):
            for slot in range(2):
                nslot = 1 - slot
                nxt_row = rbase + (kk if slot == 0 else kk + 1)
                nxt_half = 1 - slot

                if slot == 0:
                    @pl.when(kk > 0)
                    def _():
                        wait_scatter()
                else:
                    wait_scatter()

                if slot == 0:
                    issue_idx(nxt_row, nxt_half, nslot)
                    issue_gathers(nslot)
                else:
                    @pl.when(kk < ROWS // 2 - 1)
                    def _():
                        issue_idx(nxt_row, nxt_half, nslot)
                        issue_gathers(nslot)

                wait_gathers(slot)

                @pl.loop(0, CH, step=16)
                def _(qb):
                    w16 = wb[slot, pl.ds(qb, 16)]
                    for t in range(16):
                        q = qb + t
                        w = w16[t]
                        for j in range(0, L, 16):
                            va = gA[slot, q, pl.ds(j, 16)]
                            vb = gB[slot, q, pl.ds(j, 16)]
                            mbuf[q, pl.ds(j, 16)] = (
                                jnp.maximum(va + vb, F32(0.0)) * w)

                pltpu.async_copy(mbuf, agg_sh.at[db.at[slot]], ssem,
                                 add=True)

        wait_scatter()
        plsc.subcore_barrier()
        pltpu.sync_copy(agg_sh.at[pl.ds(sbase, STRIPE)],
                        out_hbm.at[cid, pl.ds(sbase, STRIPE)])

    return k(w_p, src_p, dst_p, a_nd, b_nd)


def _dot_t(a, w):
    return lax.dot_general(a, w, (((1,), (1,)), ((), ())),
                           preferred_element_type=F32,
                           precision=lax.Precision.HIGHEST)


def _enc_body(x_ref, hid_ref, we_ref, be_ref, wm_ref, bm_ref,
              h_ref, a_ref, b_ref):
    we = we_ref[...]
    h = _dot_t(x_ref[...], we[:, :L]) + _dot_t(hid_ref[...], we[:, L:])
    h = jnp.maximum(h + be_ref[...], F32(0.0))
    h_ref[...] = h
    wm = wm_ref[...]
    a_ref[...] = _dot_t(h, wm[:, :L])
    b_ref[...] = _dot_t(h, wm[:, L:]) + bm_ref[...]


def _upd_body(h_ref, p0_ref, p1_ref, wu_ref, bu_ref, o_ref):
    wu = wu_ref[...]
    agg = p0_ref[...] + p1_ref[...]
    o = _dot_t(h_ref[...], wu[:, :L]) + _dot_t(agg, wu[:, L:])
    o_ref[...] = jnp.maximum(o + bu_ref[...], F32(0.0))


_ROWBLK = 640


def _row_spec():
    return pl.BlockSpec((_ROWBLK, L), lambda i: (i, 0))


def _full_spec(shape):
    nd = len(shape)
    return pl.BlockSpec(shape, lambda i: (0,) * nd)


def _encode(xp, hp, W_enc, b_enc, W_msg, b_msg):
    grid = (NP // _ROWBLK,)
    sds = jax.ShapeDtypeStruct((NP, L), F32)
    return pl.pallas_call(
        _enc_body,
        grid=grid,
        in_specs=[_row_spec(), _row_spec(),
                  _full_spec((L, 2 * L)), _full_spec((1, L)),
                  _full_spec((L, 2 * L)), _full_spec((1, L))],
        out_specs=[_row_spec(), _row_spec(), _row_spec()],
        out_shape=[sds, sds, sds],
    )(xp, hp, W_enc, b_enc, W_msg, b_msg)


def _update(h, p0, p1, W_upd, b_upd):
    grid = (NP // _ROWBLK,)
    return pl.pallas_call(
        _upd_body,
        grid=grid,
        in_specs=[_row_spec(), _row_spec(), _row_spec(),
                  _full_spec((L, 2 * L)), _full_spec((1, L))],
        out_specs=_row_spec(),
        out_shape=jax.ShapeDtypeStruct((NP, L), F32),
    )(h, p0, p1, W_upd, b_upd)


def kernel(x, hidden, y, edge_index, W_enc, b_enc, W_msg, b_msg, W_upd, b_upd):
    src = edge_index[0].astype(I32)
    dst = edge_index[1].astype(I32)
    pad = EP - E
    pad_idx = (jnp.arange(pad, dtype=I32) * 13) % N
    src_p = jnp.concatenate([src, pad_idx])
    dst_p = jnp.concatenate([dst, pad_idx])
    y_p = jnp.concatenate([y, jnp.zeros((pad,), F32)])
    src_p = src_p.reshape(16, ROWS, 128)
    dst_p = dst_p.reshape(16, ROWS, 128)
    y_p = y_p.reshape(16, ROWS, 128)

    zpad = jnp.zeros((NP - N, L), F32)
    xp = jnp.concatenate([x, zpad])
    hp = jnp.concatenate([hidden, zpad])

    h, a_nd, b_nd = _encode(xp, hp, W_enc, b_enc.reshape(1, L),
                            W_msg, b_msg.reshape(1, L))
    w_p = _sc_sinkhorn(y_p, src_p, dst_p)
    parts = _sc_message(w_p, src_p, dst_p, a_nd, b_nd)
    h_new = _update(h, parts[0], parts[1], W_upd, b_upd.reshape(1, L))
    return h_new[:N]
